# Initial kernel scaffold; baseline (speedup 1.0000x reference)
import jax
import jax.numpy as jnp
from jax.experimental import pallas as pl


def kernel(xs, ys, cands, table):
    raise NotImplementedError("write your pallas kernel here")


# speedup vs baseline: 2.0041x; 2.0041x over previous
import functools

import jax
import jax.numpy as jnp
from jax import lax
from jax.experimental import pallas as pl
from jax.experimental.pallas import tpu as pltpu
from jax.experimental.pallas import tpu_sc as plsc

_VOCAB = 100000
_D = 64
_NORM_CAP = 10.0
_SEQ = 50
_SEQ_PAD = 56
_NC, _NS = 2, 16
_NW = _NC * _NS
_NSEQ = 22 * 1024
_SEQ_PER_W = _NSEQ // _NW
_G = 8
_CHUNKS = _SEQ_PER_W // _G
_IDX_PER_CHUNK = _G * _SEQ_PAD
_GATHER_SPLIT = 112


def _norm_body(t_ref, o_ref):
    x = t_ref[...]
    ss = jnp.sum(x * x, axis=1, keepdims=True)
    norm = jnp.sqrt(ss)
    scale = jnp.minimum(1.0, _NORM_CAP / jnp.maximum(norm, 1e-7))
    o_ref[...] = x * scale


def _normalize_table(table):
    blk = 2000
    return pl.pallas_call(
        _norm_body,
        grid=(_VOCAB // blk,),
        in_specs=[pl.BlockSpec((blk, _D), lambda i: (i, 0))],
        out_specs=pl.BlockSpec((blk, _D), lambda i: (i, 0)),
        out_shape=jax.ShapeDtypeStruct((_VOCAB, _D), jnp.float32),
    )(table)


def _pool_body(table_hbm, idx_hbm, out_hbm, idx_v, rows_v, out_v, sem):
    c = lax.axis_index("c")
    s = lax.axis_index("s")
    wid = s * _NC + c

    def chunk(g, carry):
        seq_base = wid * _SEQ_PER_W + g * _G
        pltpu.sync_copy(idx_hbm.at[pl.ds(seq_base * _SEQ_PAD, _IDX_PER_CHUNK)],
                        idx_v)
        copies = []
        for p in range(_IDX_PER_CHUNK // _GATHER_SPLIT):
            copies.append(pltpu.async_copy(
                table_hbm.at[idx_v.at[pl.ds(p * _GATHER_SPLIT, _GATHER_SPLIT)]],
                rows_v.at[pl.ds(p * _GATHER_SPLIT, _GATHER_SPLIT)],
                sem))
        for cp in copies:
            cp.wait()
        inv = jnp.float32(1.0 / _SEQ)
        for q in range(_G):
            zero = jnp.zeros((16,), jnp.float32)

            def racc(t, acc, q=q):
                res = list(acc)
                for k in range(8):
                    row = q * _SEQ_PAD + t * 8 + k
                    for j in range(4):
                        res[j] = res[j] + rows_v[row, pl.ds(j * 16, 16)]
                return tuple(res)

            acc = lax.fori_loop(0, _SEQ_PAD // 8, racc, (zero,) * 4)
            for j in range(4):
                out_v[pl.ds(q * _D + j * 16, 16)] = acc[j] * inv
        pltpu.sync_copy(out_v, out_hbm.at[pl.ds(seq_base * _D, _G * _D)])
        return carry

    lax.fori_loop(0, _CHUNKS, chunk, 0)


def _pool(table_n, idx_flat):
    mesh = plsc.VectorSubcoreMesh(core_axis_name="c", subcore_axis_name="s")
    fn = pl.kernel(
        _pool_body,
        out_type=jax.ShapeDtypeStruct((_NSEQ * _D,), jnp.float32),
        mesh=mesh,
        scratch_types=[
            pltpu.VMEM((_IDX_PER_CHUNK,), jnp.int32),
            pltpu.VMEM((_IDX_PER_CHUNK, _D), jnp.float32),
            pltpu.VMEM((_G * _D,), jnp.float32),
            pltpu.SemaphoreType.DMA,
        ],
        compiler_params=pltpu.CompilerParams(use_tc_tiling_on_sc=False),
    )
    return fn(table_n, idx_flat)


def kernel(xs, ys, cands, table):
    xs = xs.astype(jnp.int32)
    ys = ys.astype(jnp.int32)
    cands = cands.astype(jnp.int32)
    n_cands = cands.shape[0]
    idx = jnp.concatenate(
        [xs, ys, cands.reshape(n_cands * cands.shape[1], _SEQ)], axis=0)
    idx = jnp.pad(idx, ((0, 0), (0, _SEQ_PAD - _SEQ)))
    table_n = _normalize_table(table)
    pooled = _pool(table_n, idx.reshape(-1)).reshape(_NSEQ, _D)
    xs_emb = pooled[:1024]
    rest = pooled[1024:]
    xs_enc = jnp.broadcast_to(xs_emb[None], (1 + n_cands, 1024, _D))
    return (xs_enc.reshape(-1, _D), rest)
